# baseline (speedup 1.0000x reference)
import functools

import jax
import jax.numpy as jnp
from jax import lax
from jax.experimental import pallas as pl
from jax.experimental.pallas import tpu as pltpu
from jax.experimental.pallas import tpu_sc as plsc

DIM = 64
N_EMBED = 8192
N_TOK = 16 * 1024
BETA = 0.25

TM = 512
N_I = N_TOK // TM


def _dist_body(x_ref, embed_ref, w_ref, b_ref,
               d_ref, idx_ref, dsum_ref, qcbp_ref,
               acc, qcb_s, cn_s, colf_s):
    i = pl.program_id(0)

    @pl.when(i == 0)
    def _project():
        colf_s[...] = jax.lax.broadcasted_iota(
            jnp.int32, (1, N_EMBED), 1).astype(jnp.float32)
        qcb = (lax.dot_general(embed_ref[...], w_ref[...],
                               (((1,), (1,)), ((), ())))
               + b_ref[...][None, :])
        qcb_s[...] = qcb
        qcbp_ref[:, :DIM] = qcb
        qcbp_ref[:, DIM:] = jnp.zeros((N_EMBED, DIM), jnp.float32)
        sumc = jnp.sum(qcb * qcb, axis=1, keepdims=True)
        cn_s[...] = sumc.T

    x = x_ref[0]
    sumx = jnp.sum(x * x, axis=1, keepdims=True)
    dot2 = lax.dot_general(x + x, qcb_s[...],
                           (((1,), (1,)), ((), ())))
    d = (sumx + cn_s[...]) - dot2
    d_ref[...] = d

    m = jnp.min(d, axis=1, keepdims=True)
    idxf = jnp.min(jnp.where(d == m, colf_s[...], jnp.float32(N_EMBED)),
                   axis=1, keepdims=True)
    idx_ref[...] = idxf.astype(jnp.int32)
    tile_sum = jnp.sum(m)

    @pl.when(i == 0)
    def _first():
        acc[0] = tile_sum

    @pl.when(i > 0)
    def _rest():
        acc[0] = acc[0] + tile_sum

    dsum_ref[0, 0] = acc[0]


def _distance_argmin(input, embed, proj_w, proj_b):
    return pl.pallas_call(
        _dist_body,
        grid=(N_I,),
        in_specs=[
            pl.BlockSpec((1, TM, DIM), lambda i: (i // (1024 // TM),
                                                  i % (1024 // TM), 0)),
            pl.BlockSpec((N_EMBED, DIM), lambda i: (0, 0)),
            pl.BlockSpec((DIM, DIM), lambda i: (0, 0)),
            pl.BlockSpec((DIM,), lambda i: (0,)),
        ],
        out_specs=[
            pl.BlockSpec((TM, N_EMBED), lambda i: (i, 0)),
            pl.BlockSpec((TM, 1), lambda i: (i, 0)),
            pl.BlockSpec((1, 1), lambda i: (0, 0),
                         memory_space=pltpu.SMEM),
            pl.BlockSpec((N_EMBED, 2 * DIM), lambda i: (0, 0)),
        ],
        out_shape=[
            jax.ShapeDtypeStruct((N_TOK, N_EMBED), jnp.float32),
            jax.ShapeDtypeStruct((N_TOK, 1), jnp.int32),
            jax.ShapeDtypeStruct((1, 1), jnp.float32),
            jax.ShapeDtypeStruct((N_EMBED, 2 * DIM), jnp.float32),
        ],
        scratch_shapes=[
            pltpu.SMEM((1,), jnp.float32),
            pltpu.VMEM((N_EMBED, DIM), jnp.float32),
            pltpu.VMEM((1, N_EMBED), jnp.float32),
            pltpu.VMEM((1, N_EMBED), jnp.float32),
        ],
        compiler_params=pltpu.CompilerParams(
            allow_input_fusion=[True, True, True, True]),
    )(input, embed, proj_w, proj_b)


def _make_sc_gather():
    info = plsc.get_sparse_core_info()
    nw = info.num_cores * info.num_subcores
    b_per_w = N_TOK // nw
    mesh = plsc.VectorSubcoreMesh(core_axis_name="c", subcore_axis_name="s")

    @functools.partial(
        pl.kernel, mesh=mesh,
        out_type=jax.ShapeDtypeStruct((N_TOK, 2 * DIM), jnp.float32),
        scratch_types=[
            pltpu.VMEM((b_per_w,), jnp.int32),
            pltpu.VMEM((b_per_w, 2 * DIM), jnp.float32),
            pltpu.SemaphoreType.DMA,
        ],
    )
    def gather(table_hbm, idx_hbm, out_hbm, idx_v, rows_v, sem):
        wid = lax.axis_index("s") * info.num_cores + lax.axis_index("c")
        base = wid * b_per_w
        pltpu.sync_copy(idx_hbm.at[pl.ds(base, b_per_w)], idx_v)
        pltpu.async_copy(table_hbm.at[idx_v], rows_v, sem).wait()
        pltpu.sync_copy(rows_v, out_hbm.at[pl.ds(base, b_per_w)])

    return gather


def kernel(input, is_look_back, embed, proj_w, proj_b):
    d, idx, dsum, qcbp = _distance_argmin(input, embed, proj_w, proj_b)
    zq = _make_sc_gather()(qcbp, idx.reshape(-1))
    z_quantize = zq[:, :DIM].reshape(input.shape)
    diff = (1.0 + BETA) * dsum[0, 0] / jnp.float32(N_TOK * DIM)
    embed_ind = idx.reshape(input.shape[:-1])
    return (z_quantize, diff, embed_ind, d)


# speedup vs baseline: 1.0046x; 1.0046x over previous
import functools

import jax
import jax.numpy as jnp
from jax import lax
from jax.experimental import pallas as pl
from jax.experimental.pallas import tpu as pltpu
from jax.experimental.pallas import tpu_sc as plsc

DIM = 64
N_EMBED = 8192
N_TOK = 16 * 1024
BETA = 0.25

TM = 512
N_I = N_TOK // TM


def _dist_body(x_ref, embed_ref, w_ref, b_ref,
               d_ref, idx_ref, dsum_ref, qcbp_ref,
               acc, qcb_s, cn_s, colf_s):
    i = pl.program_id(0)

    @pl.when(i == 0)
    def _project():
        colf_s[...] = jax.lax.broadcasted_iota(
            jnp.int32, (1, N_EMBED), 1).astype(jnp.float32)
        qcb = (lax.dot_general(embed_ref[...], w_ref[...],
                               (((1,), (1,)), ((), ())))
               + b_ref[...][None, :])
        qcb_s[...] = qcb
        qcbp_ref[:, :DIM] = qcb
        qcbp_ref[:, DIM:] = jnp.zeros((N_EMBED, DIM), jnp.float32)
        sumc = jnp.sum(qcb * qcb, axis=1, keepdims=True)
        cn_s[...] = sumc.T

    x = x_ref[0]
    sumx = jnp.sum(x * x, axis=1, keepdims=True)
    dot2 = lax.dot_general(x + x, qcb_s[...],
                           (((1,), (1,)), ((), ())))
    d = (sumx + cn_s[...]) - dot2
    d_ref[...] = d

    m = jnp.min(d, axis=1, keepdims=True)
    idxf = jnp.min(jnp.where(d == m, colf_s[...], jnp.float32(N_EMBED)),
                   axis=1, keepdims=True)
    idx_ref[...] = idxf.astype(jnp.int32)
    tile_sum = jnp.sum(m)

    @pl.when(i == 0)
    def _first():
        acc[0] = tile_sum

    @pl.when(i > 0)
    def _rest():
        acc[0] = acc[0] + tile_sum

    dsum_ref[0, 0] = acc[0]


def _distance_argmin(input, embed, proj_w, proj_b):
    return pl.pallas_call(
        _dist_body,
        grid=(N_I,),
        in_specs=[
            pl.BlockSpec((1, TM, DIM), lambda i: (i // (1024 // TM),
                                                  i % (1024 // TM), 0)),
            pl.BlockSpec((N_EMBED, DIM), lambda i: (0, 0)),
            pl.BlockSpec((DIM, DIM), lambda i: (0, 0)),
            pl.BlockSpec((DIM,), lambda i: (0,)),
        ],
        out_specs=[
            pl.BlockSpec((TM, N_EMBED), lambda i: (i, 0)),
            pl.BlockSpec((TM, 1), lambda i: (i, 0)),
            pl.BlockSpec((1, 1), lambda i: (0, 0),
                         memory_space=pltpu.SMEM),
            pl.BlockSpec((N_EMBED, 2 * DIM), lambda i: (0, 0)),
        ],
        out_shape=[
            jax.ShapeDtypeStruct((N_TOK, N_EMBED), jnp.float32),
            jax.ShapeDtypeStruct((N_TOK, 1), jnp.int32),
            jax.ShapeDtypeStruct((1, 1), jnp.float32),
            jax.ShapeDtypeStruct((N_EMBED, 2 * DIM), jnp.float32),
        ],
        scratch_shapes=[
            pltpu.SMEM((1,), jnp.float32),
            pltpu.VMEM((N_EMBED, DIM), jnp.float32),
            pltpu.VMEM((1, N_EMBED), jnp.float32),
            pltpu.VMEM((1, N_EMBED), jnp.float32),
        ],
    )(input, embed, proj_w, proj_b)


def _make_sc_gather():
    info = plsc.get_sparse_core_info()
    nw = info.num_cores * info.num_subcores
    b_per_w = N_TOK // nw
    mesh = plsc.VectorSubcoreMesh(core_axis_name="c", subcore_axis_name="s")

    @functools.partial(
        pl.kernel, mesh=mesh,
        out_type=jax.ShapeDtypeStruct((N_TOK, 2 * DIM), jnp.float32),
        scratch_types=[
            pltpu.VMEM((b_per_w,), jnp.int32),
            pltpu.VMEM((b_per_w, 2 * DIM), jnp.float32),
            pltpu.SemaphoreType.DMA,
        ],
    )
    def gather(table_hbm, idx_hbm, out_hbm, idx_v, rows_v, sem):
        wid = lax.axis_index("s") * info.num_cores + lax.axis_index("c")
        base = wid * b_per_w
        pltpu.sync_copy(idx_hbm.at[pl.ds(base, b_per_w)], idx_v)
        pltpu.async_copy(table_hbm.at[idx_v], rows_v, sem).wait()
        pltpu.sync_copy(rows_v, out_hbm.at[pl.ds(base, b_per_w)])

    return gather


def kernel(input, is_look_back, embed, proj_w, proj_b):
    d, idx, dsum, qcbp = _distance_argmin(input, embed, proj_w, proj_b)
    zq = _make_sc_gather()(qcbp, idx.reshape(-1))
    z_quantize = zq[:, :DIM].reshape(input.shape)
    diff = (1.0 + BETA) * dsum[0, 0] / jnp.float32(N_TOK * DIM)
    embed_ind = idx.reshape(input.shape[:-1])
    return (z_quantize, diff, embed_ind, d)
